# baseline (speedup 1.0000x reference)
import jax
import jax.numpy as jnp
from jax.experimental import pallas as pl
from jax.experimental.pallas import tpu as pltpu

_BN_EPS = 1e-5
_SIN = 6
_SOUT = 6


def _make_body(B, C, HW):
    nb = B // 2

    def body(x_hbm, wp_ref, bp_ref, vo_ref, o_hbm,
             in_bufs, out_bufs, in_sems, out_sems):
        core = pl.program_id(0)
        base = core * nb

        for k in range(min(_SIN, nb)):
            pltpu.make_async_copy(
                x_hbm.at[base + k], in_bufs.at[k], in_sems.at[k]).start()

        for i in range(nb):
            si = i % _SIN
            so = i % _SOUT
            if i >= _SOUT:
                pltpu.make_async_copy(
                    out_bufs.at[so], out_bufs.at[so], out_sems.at[so]).wait()
            pltpu.make_async_copy(
                in_bufs.at[si], in_bufs.at[si], in_sems.at[si]).wait()

            x = in_bufs[si]
            p = jnp.dot(wp_ref[...], x,
                        preferred_element_type=jnp.float32) + bp_ref[...]
            s = jnp.sum(p[2:3, :] * p[0:1, :], axis=1, keepdims=True)
            y = p[1:2, :] * s
            u = jnp.concatenate(
                [y, jnp.ones((1, HW), jnp.float32),
                 jnp.zeros((6, HW), jnp.float32)], axis=0)
            out_bufs[so] = x + jnp.dot(vo_ref[...], u,
                                       preferred_element_type=jnp.float32)

            j = i + _SIN
            if j < nb:
                pltpu.make_async_copy(
                    x_hbm.at[base + j], in_bufs.at[si], in_sems.at[si]).start()
            pltpu.make_async_copy(
                out_bufs.at[so], o_hbm.at[base + i], out_sems.at[so]).start()

        for k in range(max(0, nb - _SOUT), nb):
            so = k % _SOUT
            pltpu.make_async_copy(
                out_bufs.at[so], out_bufs.at[so], out_sems.at[so]).wait()

    return body


def kernel(x, g_w, g_b, theta_w, theta_b, phi_w, phi_b,
           W_w, W_b, bn_gamma, bn_beta, bn_mean, bn_var):
    B, C, H, W = x.shape
    HW = H * W
    x_chw = x.reshape(B, C, HW)
    inv_hw = jnp.float32(1.0 / HW)

    f32 = jnp.float32
    wp = jnp.zeros((8, C), f32)
    wp = wp.at[0, :].set(g_w.astype(f32))
    wp = wp.at[1, :].set(theta_w.astype(f32))
    wp = wp.at[2, :].set(phi_w.astype(f32) * inv_hw)
    bp = jnp.zeros((8, 1), f32)
    bp = bp.at[0, 0].set(g_b[0].astype(f32))
    bp = bp.at[1, 0].set(theta_b[0].astype(f32))
    bp = bp.at[2, 0].set(phi_b[0].astype(f32) * inv_hw)

    inv_std = jax.lax.rsqrt(bn_var.astype(f32) + _BN_EPS)
    scale = bn_gamma.astype(f32) * inv_std
    weff = W_w.astype(f32) * scale
    beff = W_b.astype(f32) * scale + bn_beta.astype(f32) - bn_mean.astype(f32) * scale
    vo = jnp.zeros((C, 8), f32)
    vo = vo.at[:, 0].set(weff)
    vo = vo.at[:, 1].set(beff)

    out_chw = pl.pallas_call(
        _make_body(B, C, HW),
        out_shape=jax.ShapeDtypeStruct((B, C, HW), x.dtype),
        grid=(2,),
        in_specs=[
            pl.BlockSpec(memory_space=pl.ANY),
            pl.BlockSpec((8, C), lambda c: (0, 0)),
            pl.BlockSpec((8, 1), lambda c: (0, 0)),
            pl.BlockSpec((C, 8), lambda c: (0, 0)),
        ],
        out_specs=pl.BlockSpec(memory_space=pl.ANY),
        scratch_shapes=[
            pltpu.VMEM((_SIN, C, HW), f32),
            pltpu.VMEM((_SOUT, C, HW), f32),
            pltpu.SemaphoreType.DMA((_SIN,)),
            pltpu.SemaphoreType.DMA((_SOUT,)),
        ],
        compiler_params=pltpu.CompilerParams(
            dimension_semantics=("parallel",)),
    )(x_chw, wp, bp, vo)

    return out_chw.reshape(B, C, H, W)
